# baseline (speedup 1.0000x reference)
import functools

import jax
import jax.numpy as jnp
from jax import lax
from jax.experimental import pallas as pl
from jax.experimental.pallas import tpu as pltpu
from jax.experimental.pallas import tpu_sc as plsc

B = 4096
N = 50
D = 64
NQ = 4
NN = 20
NF = NQ * NN
BN = B * N

NC = 2
NS = 16
NW = NC * NS
NROW = 64
PAIR = 2
STEP_ROWS = PAIR * NROW
STEPS_PER_W = B // NW // PAIR
NBUF = 5


def _table_body(q_ref, n_ref, w1_ref, b1_ref, w2_ref, b2_ref, f_ref):
    rq = lax.broadcasted_iota(jnp.int32, (NF, NQ), 0) // NN
    cq = lax.broadcasted_iota(jnp.int32, (NF, NQ), 1)
    oh_q = jnp.where(rq == cq, 1.0, 0.0).astype(jnp.float32)
    rn = lax.broadcasted_iota(jnp.int32, (NF, NN), 0) % NN
    cn = lax.broadcasted_iota(jnp.int32, (NF, NN), 1)
    oh_n = jnp.where(rn == cn, 1.0, 0.0).astype(jnp.float32)
    qe = jnp.dot(oh_q, q_ref[...], preferred_element_type=jnp.float32)
    ne = jnp.dot(oh_n, n_ref[...], preferred_element_type=jnp.float32)
    combined = jnp.concatenate([qe, ne * qe], axis=-1)
    h = jnp.maximum(
        jnp.dot(combined, w1_ref[...], preferred_element_type=jnp.float32)
        + b1_ref[...],
        0.0,
    )
    res = jnp.dot(h, w2_ref[...], preferred_element_type=jnp.float32) + b2_ref[...]
    f_ref[...] = jnp.concatenate(
        [res, jnp.zeros((NF, 128 - D), jnp.float32)], axis=-1
    )


_table_call = pl.pallas_call(
    _table_body,
    out_shape=jax.ShapeDtypeStruct((NF, 128), jnp.float32),
)


@functools.cache
def _build_sc_gather():
    @functools.partial(
        pl.kernel,
        mesh=plsc.VectorSubcoreMesh(core_axis_name="c", subcore_axis_name="s"),
        out_type=jax.ShapeDtypeStruct((B // PAIR, STEP_ROWS, 128), jnp.float32),
        scratch_types=[
            pltpu.VMEM((STEPS_PER_W, STEP_ROWS), jnp.int32),
            pltpu.VMEM((STEPS_PER_W, STEP_ROWS), jnp.int32),
            pltpu.VMEM((STEPS_PER_W, STEP_ROWS), jnp.int32),
            pltpu.VMEM((NBUF, STEP_ROWS, 128), jnp.float32),
            pltpu.SemaphoreType.DMA((NBUF,)),
            pltpu.SemaphoreType.DMA((NBUF,)),
        ],
    )
    def _sc_gather(f_hbm, q_hbm, n_hbm, out_hbm, q_v, n_v, idx_v, rows_v,
                   gsem, ssem):
        wid = lax.axis_index("s") * NC + lax.axis_index("c")
        pltpu.sync_copy(q_hbm.at[wid], q_v)
        pltpu.sync_copy(n_hbm.at[wid], n_v)

        tbase = wid * NF

        def fuse_body(j, carry):
            for c in range(STEP_ROWS // 16):
                s = pl.ds(c * 16, 16)
                idx_v[j, s] = q_v[j, s] * NN + n_v[j, s] + tbase
            return carry

        lax.fori_loop(0, STEPS_PER_W, fuse_body, 0)

        bbase = wid * STEPS_PER_W

        def gather(j, b):
            pltpu.async_copy(f_hbm.at[idx_v.at[j]], rows_v.at[b], gsem.at[b])

        def wait_gather(b):
            pltpu.make_async_copy(
                f_hbm.at[idx_v.at[0]], rows_v.at[b], gsem.at[b]
            ).wait()

        def scatter(j, b):
            pltpu.async_copy(rows_v.at[b], out_hbm.at[bbase + j], ssem.at[b])

        def wait_scatter(b):
            pltpu.make_async_copy(
                rows_v.at[b], out_hbm.at[bbase], ssem.at[b]
            ).wait()

        AHEAD = 3
        for b in range(AHEAD):
            gather(b, b)

        for j in range(STEPS_PER_W):
            b = j % NBUF
            wait_gather(b)
            scatter(j, b)
            jn = j + AHEAD
            if jn < STEPS_PER_W:
                bn = jn % NBUF
                if jn >= NBUF:
                    wait_scatter(bn)
                gather(jn, bn)

        for b in range(NBUF):
            wait_scatter(b)

    return _sc_gather


def kernel(query_type, node_operator, Q_table, N_table, W1, b1, W2, b2):
    fused_table = _table_call(
        Q_table, N_table, W1, b1.reshape(1, D), W2, b2.reshape(1, D)
    )
    fused_table = jnp.broadcast_to(fused_table[None], (NW, NF, 128)).reshape(
        NW * NF, 128
    )
    if query_type.dtype != jnp.int32:
        query_type = query_type.astype(jnp.int32)
    if node_operator.dtype != jnp.int32:
        node_operator = node_operator.astype(jnp.int32)
    qp = jnp.pad(query_type, ((0, 0), (0, NROW - N))).reshape(
        NW, STEPS_PER_W, STEP_ROWS
    )
    np_ = jnp.pad(node_operator, ((0, 0), (0, NROW - N))).reshape(
        NW, STEPS_PER_W, STEP_ROWS
    )
    out = _build_sc_gather()(fused_table, qp, np_)
    return out.reshape(B, NROW, 128)[:, :N, :D]


# speedup vs baseline: 2.3432x; 2.3432x over previous
import functools

import jax
import jax.numpy as jnp
from jax import lax
from jax.experimental import pallas as pl
from jax.experimental.pallas import tpu as pltpu
from jax.experimental.pallas import tpu_sc as plsc

B = 4096
N = 50
D = 64
NQ = 4
NN = 20
NF = NQ * NN
BN = B * N

NC = 2
NS = 16
NW = NC * NS
NROW = 56
PAIR = 2
STEP_ROWS = PAIR * NROW
STEPS_PER_W = B // NW // PAIR
NBUF = 6


def _table_body(q_ref, n_ref, w1_ref, b1_ref, w2_ref, b2_ref, f_ref):
    rq = lax.broadcasted_iota(jnp.int32, (NF, NQ), 0) // NN
    cq = lax.broadcasted_iota(jnp.int32, (NF, NQ), 1)
    oh_q = jnp.where(rq == cq, 1.0, 0.0).astype(jnp.float32)
    rn = lax.broadcasted_iota(jnp.int32, (NF, NN), 0) % NN
    cn = lax.broadcasted_iota(jnp.int32, (NF, NN), 1)
    oh_n = jnp.where(rn == cn, 1.0, 0.0).astype(jnp.float32)
    qe = jnp.dot(oh_q, q_ref[...], preferred_element_type=jnp.float32)
    ne = jnp.dot(oh_n, n_ref[...], preferred_element_type=jnp.float32)
    combined = jnp.concatenate([qe, ne * qe], axis=-1)
    h = jnp.maximum(
        jnp.dot(combined, w1_ref[...], preferred_element_type=jnp.float32)
        + b1_ref[...],
        0.0,
    )
    res = jnp.dot(h, w2_ref[...], preferred_element_type=jnp.float32) + b2_ref[...]
    f_ref[...] = jnp.concatenate(
        [res, jnp.zeros((NF, 128 - D), jnp.float32)], axis=-1
    )


_table_call = pl.pallas_call(
    _table_body,
    out_shape=jax.ShapeDtypeStruct((NF, 128), jnp.float32),
)


@functools.cache
def _build_sc_gather():
    @functools.partial(
        pl.kernel,
        mesh=plsc.VectorSubcoreMesh(core_axis_name="c", subcore_axis_name="s"),
        out_type=jax.ShapeDtypeStruct((B // PAIR, STEP_ROWS, 128), jnp.float32),
        scratch_types=[
            pltpu.VMEM((STEPS_PER_W, STEP_ROWS), jnp.int32),
            pltpu.VMEM((STEPS_PER_W, STEP_ROWS), jnp.int32),
            pltpu.VMEM((STEPS_PER_W, STEP_ROWS), jnp.int32),
            pltpu.VMEM((NBUF, STEP_ROWS, 128), jnp.float32),
            pltpu.SemaphoreType.DMA((NBUF,)),
            pltpu.SemaphoreType.DMA((NBUF,)),
        ],
    )
    def _sc_gather(f_hbm, q_hbm, n_hbm, out_hbm, q_v, n_v, idx_v, rows_v,
                   gsem, ssem):
        wid = lax.axis_index("s") * NC + lax.axis_index("c")
        pltpu.sync_copy(q_hbm.at[wid], q_v)
        pltpu.sync_copy(n_hbm.at[wid], n_v)

        tbase = wid * NF

        def fuse_body(j, carry):
            for c in range(STEP_ROWS // 16):
                s = pl.ds(c * 16, 16)
                idx_v[j, s] = q_v[j, s] * NN + n_v[j, s] + tbase
            return carry

        lax.fori_loop(0, STEPS_PER_W, fuse_body, 0)

        bbase = wid * STEPS_PER_W

        def gather(j, b):
            pltpu.async_copy(f_hbm.at[idx_v.at[j]], rows_v.at[b], gsem.at[b])

        def wait_gather(b):
            pltpu.make_async_copy(
                f_hbm.at[idx_v.at[0]], rows_v.at[b], gsem.at[b]
            ).wait()

        def scatter(j, b):
            pltpu.async_copy(rows_v.at[b], out_hbm.at[bbase + j], ssem.at[b])

        def wait_scatter(b):
            pltpu.make_async_copy(
                rows_v.at[b], out_hbm.at[bbase], ssem.at[b]
            ).wait()

        AHEAD = 3
        for b in range(AHEAD):
            gather(b, b)

        for j in range(STEPS_PER_W):
            b = j % NBUF
            wait_gather(b)
            scatter(j, b)
            jn = j + AHEAD
            if jn < STEPS_PER_W:
                bn = jn % NBUF
                if jn >= NBUF:
                    wait_scatter(bn)
                gather(jn, bn)

        for b in range(NBUF):
            wait_scatter(b)

    return _sc_gather


def kernel(query_type, node_operator, Q_table, N_table, W1, b1, W2, b2):
    fused_table = _table_call(
        Q_table, N_table, W1, b1.reshape(1, D), W2, b2.reshape(1, D)
    )
    fused_table = jnp.broadcast_to(fused_table[None], (NW, NF, 128)).reshape(
        NW * NF, 128
    )
    if query_type.dtype != jnp.int32:
        query_type = query_type.astype(jnp.int32)
    if node_operator.dtype != jnp.int32:
        node_operator = node_operator.astype(jnp.int32)
    spread = jnp.arange(B)[:, None] * (NROW - N) + jnp.arange(NROW - N)[None, :]
    qp = jnp.concatenate(
        [query_type, (spread % NF) // NN], axis=1
    ).reshape(NW, STEPS_PER_W, STEP_ROWS)
    np_ = jnp.concatenate(
        [node_operator, spread % NN], axis=1
    ).reshape(NW, STEPS_PER_W, STEP_ROWS)
    out = _build_sc_gather()(fused_table, qp, np_)
    return out.reshape(B, NROW, 128)[:, :N, :D]
